# Initial kernel scaffold; baseline (speedup 1.0000x reference)
import jax
import jax.numpy as jnp
from jax.experimental import pallas as pl


def kernel(sampled_points, sampled_x, edge_index_filtered, Wc, bc, Wo, bo, Wq, bq, Wk, bk):
    raise NotImplementedError("write your pallas kernel here")


# speedup vs baseline: 1.9126x; 1.9126x over previous
import functools

import jax
import jax.numpy as jnp
from jax import lax
from jax.experimental import pallas as pl
from jax.experimental.pallas import tpu as pltpu
from jax.experimental.pallas import tpu_sc as plsc

N = 10000
E = 320000
H = 128
F = 64
K = 15
SLOTS = 16
NBLK = N // 16
NBLKP = 640
BPW = NBLKP // 32
NSL = 8
FSL = F // NSL
QROW = 1152
_SCALE = 8.000001


def _proj_body(x_ref, p_ref, wq_ref, bq_ref, wk_ref, bk_ref, wc_ref,
               bc_ref, wo_ref, bo_ref, qt_ref, kall_ref, g4_ref, ht4_ref):
    x = x_ref[...]
    p = p_ref[...]
    wq = wq_ref[...]
    wk = wk_ref[...]
    wc = wc_ref[...]
    bc = bc_ref[...]
    wo = wo_ref[...]
    bo = bo_ref[...]

    dn = (((1,), (1,)), ((), ()))
    qt = lax.dot_general(wq, x, dn) + bq_ref[...][:, None]
    qt_ref[...] = qt * (1.0 / _SCALE)
    kall_ref[...] = lax.dot_general(x, wk, dn) + bk_ref[...][None, :]

    wo4 = jnp.concatenate([wo, jnp.zeros((1, H + 3), jnp.float32)], axis=0)
    wowc4 = lax.dot_general(wo4, wc, (((1,), (0,)), ((), ())))
    a4 = wowc4[:, :H]
    b4 = wowc4[:, H:]
    bo4 = jnp.concatenate([bo, jnp.zeros((1,), jnp.float32)], axis=0)
    c4 = jnp.sum(wo4 * bc[None, :], axis=1) + bo4

    g4_ref[...] = lax.dot_general(x, a4, dn) + lax.dot_general(p, b4, dn)

    eye43 = (lax.broadcasted_iota(jnp.int32, (4, 3), 0)
             == lax.broadcasted_iota(jnp.int32, (4, 3), 1)).astype(jnp.float32)
    ht4_ref[...] = lax.dot_general(eye43 - b4, p, dn) + c4[:, None]


def _projections(p, x, wq, bq, wk, bk, wc, bc, wo, bo):
    return pl.pallas_call(
        _proj_body,
        out_shape=[
            jax.ShapeDtypeStruct((F, N), jnp.float32),
            jax.ShapeDtypeStruct((N, F), jnp.float32),
            jax.ShapeDtypeStruct((N, 4), jnp.float32),
            jax.ShapeDtypeStruct((4, N), jnp.float32),
        ],
    )(x, p, wq, bq, wk, bk, wc, bc, wo, bo)


def _build_neigh(src, dst):
    perm = jnp.argsort(src, stable=True)
    sorted_dst = dst[perm]
    counts = jnp.bincount(src, length=N)
    starts = jnp.concatenate(
        [jnp.zeros((1,), counts.dtype), jnp.cumsum(counts)[:-1]])
    slot = jnp.arange(SLOTS, dtype=counts.dtype)
    pos = starts[:, None] + slot[None, :]
    valid = slot[None, :] < jnp.minimum(counts, K)[:, None]
    gathered = sorted_dst[jnp.clip(pos, 0, E - 1)]
    neigh = jnp.where(valid, gathered, 0)
    return neigh.reshape(-1).astype(jnp.int32)


def _attn_body(neigh1d, qh1, kp1, g1, out, nball, qall, buf, sacc, obuf,
               sem):
    cid = lax.axis_index("c")
    sid = lax.axis_index("s")
    wid = sid * 2 + cid
    bstart = wid * BPW

    lane = lax.iota(jnp.int32, 16)
    lanek = [lane * SLOTS + k for k in range(SLOTS)]

    pltpu.sync_copy(neigh1d.at[pl.ds(bstart * 256, BPW * 256)], nball)
    pltpu.sync_copy(qh1.at[pl.ds(bstart * QROW, BPW * QROW)], qall)

    for j in range(NSL):
        pltpu.sync_copy(kp1.at[pl.ds(j * (N * FSL), N * FSL)], buf)

        def blk_body(blk, carry, j=j):
            qoff = blk * QROW + j * (FSL * 16)
            qv = [qall[pl.ds(qoff + t * 16, 16)] for t in range(FSL)]
            for k in range(SLOTS):
                nk = plsc.load_gather(nball, [blk * 256 + lanek[k]])
                idx = nk * FSL
                acc = plsc.load_gather(buf, [idx]) * qv[0]
                for t in range(1, FSL):
                    acc = acc + plsc.load_gather(buf, [idx + t]) * qv[t]
                soff = blk * 256 + k * 16
                if j == 0:
                    sacc[pl.ds(soff, 16)] = acc
                else:
                    sacc[pl.ds(soff, 16)] = sacc[pl.ds(soff, 16)] + acc
            return carry

        lax.fori_loop(0, BPW, blk_body, 0)

    pltpu.sync_copy(g1, buf.at[pl.ds(0, N * 4)])

    def agg_body(blk, carry):
        sm = []
        m = None
        for k in range(K):
            nk = plsc.load_gather(nball, [blk * 256 + lanek[k]])
            smk = jnp.where(nk == 0, 0.0,
                            sacc[pl.ds(blk * 256 + k * 16, 16)])
            sm.append(smk)
            m = smk if m is None else jnp.maximum(m, smk)

        esum = jnp.zeros((16,), jnp.float32)
        acc0 = jnp.zeros((16,), jnp.float32)
        acc1 = jnp.zeros((16,), jnp.float32)
        acc2 = jnp.zeros((16,), jnp.float32)
        for k in range(K):
            e = jnp.exp(sm[k] - m)
            nk = plsc.load_gather(nball, [blk * 256 + lanek[k]])
            gbase = nk * 4
            gx = plsc.load_gather(buf, [gbase])
            gy = plsc.load_gather(buf, [gbase + 1])
            gz = plsc.load_gather(buf, [gbase + 2])
            esum = esum + e
            acc0 = acc0 + e * gx
            acc1 = acc1 + e * gy
            acc2 = acc2 + e * gz

        r = 1.0 / esum
        hoff = blk * QROW + 1024
        ooff = blk * 128
        obuf[pl.ds(ooff, 16)] = qall[pl.ds(hoff, 16)] + acc0 * r
        obuf[pl.ds(ooff + 16, 16)] = qall[pl.ds(hoff + 16, 16)] + acc1 * r
        obuf[pl.ds(ooff + 32, 16)] = qall[pl.ds(hoff + 32, 16)] + acc2 * r
        return carry

    lax.fori_loop(0, BPW, agg_body, 0)
    pltpu.sync_copy(obuf, out.at[pl.ds(bstart * 128, BPW * 128)])


def _attention(neigh1d, qh1, kp1, g1):
    mesh = plsc.VectorSubcoreMesh(core_axis_name="c", subcore_axis_name="s")
    kern = functools.partial(
        pl.kernel,
        out_type=jax.ShapeDtypeStruct((NBLKP * 128,), jnp.float32),
        mesh=mesh,
        compiler_params=pltpu.CompilerParams(needs_layout_passes=False),
        scratch_types=[
            pltpu.VMEM((BPW * 256,), jnp.int32),
            pltpu.VMEM((BPW * QROW,), jnp.float32),
            pltpu.VMEM((N * FSL,), jnp.float32),
            pltpu.VMEM((BPW * 256,), jnp.float32),
            pltpu.VMEM((BPW * 128,), jnp.float32),
            pltpu.SemaphoreType.DMA,
        ],
    )(_attn_body)
    return kern(neigh1d, qh1, kp1, g1)


def kernel(sampled_points, sampled_x, edge_index_filtered, Wc, bc, Wo, bo,
           Wq, bq, Wk, bk):
    src = edge_index_filtered[0]
    dst = edge_index_filtered[1]
    qt, kall, g4, ht4 = _projections(sampled_points, sampled_x, Wq, bq,
                                     Wk, bk, Wc, bc, Wo, bo)
    qb = qt.reshape(F, NBLK, 16).transpose(1, 0, 2).reshape(NBLK, F * 16)
    hb = ht4.reshape(4, NBLK, 16).transpose(1, 0, 2).reshape(NBLK, 64)
    qh = jnp.concatenate(
        [qb, hb, jnp.zeros((NBLK, 64), jnp.float32)], axis=1)
    qh1 = jnp.concatenate(
        [qh, jnp.zeros((NBLKP - NBLK, QROW), jnp.float32)], axis=0
    ).reshape(-1)
    kp1 = kall.reshape(N, NSL, FSL).transpose(1, 0, 2).reshape(-1)
    neigh1d = _build_neigh(src, dst)
    neigh1d = jnp.concatenate(
        [neigh1d, jnp.zeros(((NBLKP - NBLK) * 256,), jnp.int32)])
    out1 = _attention(neigh1d, qh1, kp1, g4.reshape(-1))
    o = out1[:NBLK * 128].reshape(NBLK, 8, 16)[:, :3, :]
    return o.transpose(0, 2, 1).reshape(N, 3)


# speedup vs baseline: 4.2451x; 2.2195x over previous
import functools

import jax
import jax.numpy as jnp
from jax import lax
from jax.experimental import pallas as pl
from jax.experimental.pallas import tpu as pltpu
from jax.experimental.pallas import tpu_sc as plsc

N = 10000
E = 320000
H = 128
F = 64
K = 15
SLOTS = 16
NBLK = N // 16
NBLKP = 640
BPW = NBLKP // 32
NSL = 8
FSL = F // NSL
QROW = 1152
NW = 32
_SCALE = 8.000001


def _proj_body(x_ref, p_ref, wq_ref, bq_ref, wk_ref, bk_ref, wc_ref,
               bc_ref, wo_ref, bo_ref, qt_ref, kall_ref, g4_ref, ht4_ref):
    x = x_ref[...]
    p = p_ref[...]
    wq = wq_ref[...]
    wk = wk_ref[...]
    wc = wc_ref[...]
    bc = bc_ref[...]
    wo = wo_ref[...]
    bo = bo_ref[...]

    dn = (((1,), (1,)), ((), ()))
    qt = lax.dot_general(wq, x, dn) + bq_ref[...][:, None]
    qt_ref[...] = qt * (1.0 / _SCALE)
    kall_ref[...] = lax.dot_general(x, wk, dn) + bk_ref[...][None, :]

    wo4 = jnp.concatenate([wo, jnp.zeros((1, H + 3), jnp.float32)], axis=0)
    wowc4 = lax.dot_general(wo4, wc, (((1,), (0,)), ((), ())))
    a4 = wowc4[:, :H]
    b4 = wowc4[:, H:]
    bo4 = jnp.concatenate([bo, jnp.zeros((1,), jnp.float32)], axis=0)
    c4 = jnp.sum(wo4 * bc[None, :], axis=1) + bo4

    g4_ref[...] = lax.dot_general(x, a4, dn) + lax.dot_general(p, b4, dn)

    eye43 = (lax.broadcasted_iota(jnp.int32, (4, 3), 0)
             == lax.broadcasted_iota(jnp.int32, (4, 3), 1)).astype(jnp.float32)
    ht4_ref[...] = lax.dot_general(eye43 - b4, p, dn) + c4[:, None]


def _projections(p, x, wq, bq, wk, bk, wc, bc, wo, bo):
    return pl.pallas_call(
        _proj_body,
        out_shape=[
            jax.ShapeDtypeStruct((F, N), jnp.float32),
            jax.ShapeDtypeStruct((N, F), jnp.float32),
            jax.ShapeDtypeStruct((N, 4), jnp.float32),
            jax.ShapeDtypeStruct((4, N), jnp.float32),
        ],
    )(x, p, wq, bq, wk, bk, wc, bc, wo, bo)


NP = 10240
EPW = E // NW
NIMG = NP * SLOTS
_SC_PARAMS = pltpu.CompilerParams(needs_layout_passes=False)
_MESH_KW = dict(core_axis_name="c", subcore_axis_name="s")


def _count_body(src1, zeros_np, cnt_all, srcv, counts):
    wid = lax.axis_index("s") * 2 + lax.axis_index("c")
    pltpu.sync_copy(zeros_np, counts)
    pltpu.sync_copy(src1.at[pl.ds(wid * EPW, EPW)], srcv)
    ones = jnp.ones((16,), jnp.int32)

    def body(i, c):
        s = srcv[pl.ds(i * 16, 16)]
        plsc.addupdate_scatter(counts, [s], ones)
        return c

    lax.fori_loop(0, EPW // 16, body, 0)
    pltpu.sync_copy(counts, cnt_all.at[pl.ds(wid * NP, NP)])


def _counts(src1, zeros_np):
    kern = functools.partial(
        pl.kernel,
        out_type=jax.ShapeDtypeStruct((NW * NP,), jnp.int32),
        mesh=plsc.VectorSubcoreMesh(**_MESH_KW),
        compiler_params=_SC_PARAMS,
        scratch_types=[
            pltpu.VMEM((EPW,), jnp.int32),
            pltpu.VMEM((NP,), jnp.int32),
        ],
    )(_count_body)
    return kern(src1, zeros_np)


def _prefix_body(cnt1, base_all, totals, cb, bb, totv):
    wid = lax.axis_index("s") * 2 + lax.axis_index("c")
    nstart = wid * 320
    for t in range(NW):
        pltpu.sync_copy(cnt1.at[pl.ds(t * NP + nstart, 320)],
                        cb.at[pl.ds(t * 320, 320)])

    def body(v, c):
        run = jnp.zeros((16,), jnp.int32)
        for t in range(NW):
            val = cb[pl.ds(t * 320 + v * 16, 16)]
            bb[pl.ds(t * 320 + v * 16, 16)] = run
            run = run + val
        totv[pl.ds(v * 16, 16)] = run
        return c

    lax.fori_loop(0, 20, body, 0)
    for t in range(NW):
        pltpu.sync_copy(bb.at[pl.ds(t * 320, 320)],
                        base_all.at[pl.ds(t * NP + nstart, 320)])
    pltpu.sync_copy(totv, totals.at[pl.ds(nstart, 320)])


def _prefix(cnt1):
    kern = functools.partial(
        pl.kernel,
        out_type=[jax.ShapeDtypeStruct((NW * NP,), jnp.int32),
                  jax.ShapeDtypeStruct((NP,), jnp.int32)],
        mesh=plsc.VectorSubcoreMesh(**_MESH_KW),
        compiler_params=_SC_PARAMS,
        scratch_types=[
            pltpu.VMEM((NW * 320,), jnp.int32),
            pltpu.VMEM((NW * 320,), jnp.int32),
            pltpu.VMEM((320,), jnp.int32),
        ],
    )(_prefix_body)
    return kern(cnt1)


def _scatter_body(src1, dst1, base_all, totals, zeros_img, img_out,
                  srcv, dstv, bases, totv, idxl, vall, idx_stage, val_stage,
                  shared_img):
    cid = lax.axis_index("c")
    sid = lax.axis_index("s")
    wid = sid * 2 + cid
    lane = lax.iota(jnp.int32, 16)

    @pl.when(sid == 0)
    def _():
        pltpu.sync_copy(zeros_img, shared_img)
    plsc.subcore_barrier()

    pltpu.sync_copy(src1.at[pl.ds(wid * EPW, EPW)], srcv)
    pltpu.sync_copy(dst1.at[pl.ds(wid * EPW, EPW)], dstv)
    pltpu.sync_copy(base_all.at[pl.ds(wid * NP, NP)], bases)
    pltpu.sync_copy(totals.at[pl.ds(wid * 320, 320)], totv)

    def ebody(i, cur):
        s = srcv[pl.ds(i * 16, 16)]
        d = dstv[pl.ds(i * 16, 16)]
        occ, last = plsc.scan_count(s)
        cnt = plsc.load_gather(bases, [s])
        rank = cnt + occ - 1
        plsc.store_scatter(bases, [s], rank + 1, mask=last)
        valid = rank < K
        flat = s * SLOTS + rank
        nvalid = jnp.sum(jnp.where(valid, 1, 0))
        plsc.store_compressed(idxl.at[pl.ds(cur, 16)], flat, mask=valid)
        plsc.store_compressed(vall.at[pl.ds(cur, 16)], d, mask=valid)
        return cur + nvalid

    cur = lax.fori_loop(0, EPW // 16, ebody, jnp.int32(0))

    def pbody(v, cur):
        node = wid * 320 + v * 16 + lane
        tmin = jnp.minimum(totv[pl.ds(v * 16, 16)], K)
        for slot in range(SLOTS):
            padm = tmin <= slot
            flat = node * SLOTS + slot
            nv = jnp.sum(jnp.where(padm, 1, 0))
            plsc.store_compressed(idxl.at[pl.ds(cur, 16)], flat, mask=padm)
            plsc.store_compressed(vall.at[pl.ds(cur, 16)],
                                  jnp.zeros((16,), jnp.int32), mask=padm)
            cur = cur + nv
        return cur

    cur = lax.fori_loop(0, 20, pbody, cur)

    z = jnp.zeros((16,), jnp.int32)
    for t in range(8):
        idxl[pl.ds(cur + t * 16, 16)] = z
        vall[pl.ds(cur + t * 16, 16)] = z
    nch = jnp.right_shift(cur + 127, 7)

    def sbody(c, carry):
        for t in range(8):
            idx_stage[pl.ds(t * 16, 16)] = idxl[pl.ds(c * 128 + t * 16, 16)]
            val_stage[pl.ds(t * 16, 16)] = vall[pl.ds(c * 128 + t * 16, 16)]
        pltpu.sync_copy(val_stage, shared_img.at[idx_stage], add=True)
        return carry

    lax.fori_loop(0, nch, sbody, 0)
    plsc.subcore_barrier()

    @pl.when(sid == 0)
    def _():
        pltpu.sync_copy(shared_img, img_out.at[pl.ds(cid * NIMG, NIMG)])


def _scatter(src1, dst1, base_all, totals, zeros_img):
    kern = functools.partial(
        pl.kernel,
        out_type=jax.ShapeDtypeStruct((2 * NIMG,), jnp.int32),
        mesh=plsc.VectorSubcoreMesh(**_MESH_KW),
        compiler_params=_SC_PARAMS,
        scratch_types=[
            pltpu.VMEM((EPW,), jnp.int32),
            pltpu.VMEM((EPW,), jnp.int32),
            pltpu.VMEM((NP,), jnp.int32),
            pltpu.VMEM((320,), jnp.int32),
            pltpu.VMEM((16384,), jnp.int32),
            pltpu.VMEM((16384,), jnp.int32),
            pltpu.VMEM((128,), jnp.int32),
            pltpu.VMEM((128,), jnp.int32),
            pltpu.VMEM_SHARED((NIMG,), jnp.int32),
        ],
    )(_scatter_body)
    return kern(src1, dst1, base_all, totals, zeros_img)


def _attn_body(img, qh1, kp1, g1, out, nball, nbb, qall, buf, sacc, obuf,
               sem):
    cid = lax.axis_index("c")
    sid = lax.axis_index("s")
    wid = sid * 2 + cid
    bstart = wid * BPW

    lane = lax.iota(jnp.int32, 16)
    lanek = [lane * SLOTS + k for k in range(SLOTS)]

    pltpu.sync_copy(img.at[pl.ds(bstart * 256, BPW * 256)], nball)
    pltpu.sync_copy(img.at[pl.ds(NIMG + bstart * 256, BPW * 256)], nbb)
    pltpu.sync_copy(qh1.at[pl.ds(bstart * QROW, BPW * QROW)], qall)

    def merge_body(v, c):
        nball[pl.ds(v * 16, 16)] = (nball[pl.ds(v * 16, 16)]
                                    + nbb[pl.ds(v * 16, 16)])
        return c

    lax.fori_loop(0, BPW * 16, merge_body, 0)

    for j in range(NSL):
        pltpu.sync_copy(kp1.at[pl.ds(j * (N * FSL), N * FSL)], buf)

        def blk_body(blk, carry, j=j):
            qoff = blk * QROW + j * (FSL * 16)
            qv = [qall[pl.ds(qoff + t * 16, 16)] for t in range(FSL)]
            for k in range(SLOTS):
                nk = plsc.load_gather(nball, [blk * 256 + lanek[k]])
                idx = nk * FSL
                acc = plsc.load_gather(buf, [idx]) * qv[0]
                for t in range(1, FSL):
                    acc = acc + plsc.load_gather(buf, [idx + t]) * qv[t]
                soff = blk * 256 + k * 16
                if j == 0:
                    sacc[pl.ds(soff, 16)] = acc
                else:
                    sacc[pl.ds(soff, 16)] = sacc[pl.ds(soff, 16)] + acc
            return carry

        lax.fori_loop(0, BPW, blk_body, 0)

    pltpu.sync_copy(g1, buf.at[pl.ds(0, N * 4)])

    def agg_body(blk, carry):
        sm = []
        m = None
        for k in range(K):
            nk = plsc.load_gather(nball, [blk * 256 + lanek[k]])
            smk = jnp.where(nk == 0, 0.0,
                            sacc[pl.ds(blk * 256 + k * 16, 16)])
            sm.append(smk)
            m = smk if m is None else jnp.maximum(m, smk)

        esum = jnp.zeros((16,), jnp.float32)
        acc0 = jnp.zeros((16,), jnp.float32)
        acc1 = jnp.zeros((16,), jnp.float32)
        acc2 = jnp.zeros((16,), jnp.float32)
        for k in range(K):
            e = jnp.exp(sm[k] - m)
            nk = plsc.load_gather(nball, [blk * 256 + lanek[k]])
            gbase = nk * 4
            gx = plsc.load_gather(buf, [gbase])
            gy = plsc.load_gather(buf, [gbase + 1])
            gz = plsc.load_gather(buf, [gbase + 2])
            esum = esum + e
            acc0 = acc0 + e * gx
            acc1 = acc1 + e * gy
            acc2 = acc2 + e * gz

        r = 1.0 / esum
        hoff = blk * QROW + 1024
        ooff = blk * 128
        obuf[pl.ds(ooff, 16)] = qall[pl.ds(hoff, 16)] + acc0 * r
        obuf[pl.ds(ooff + 16, 16)] = qall[pl.ds(hoff + 16, 16)] + acc1 * r
        obuf[pl.ds(ooff + 32, 16)] = qall[pl.ds(hoff + 32, 16)] + acc2 * r
        return carry

    lax.fori_loop(0, BPW, agg_body, 0)
    pltpu.sync_copy(obuf, out.at[pl.ds(bstart * 128, BPW * 128)])


def _attention(img, qh1, kp1, g1):
    kern = functools.partial(
        pl.kernel,
        out_type=jax.ShapeDtypeStruct((NBLKP * 128,), jnp.float32),
        mesh=plsc.VectorSubcoreMesh(**_MESH_KW),
        compiler_params=_SC_PARAMS,
        scratch_types=[
            pltpu.VMEM((BPW * 256,), jnp.int32),
            pltpu.VMEM((BPW * 256,), jnp.int32),
            pltpu.VMEM((BPW * QROW,), jnp.float32),
            pltpu.VMEM((N * FSL,), jnp.float32),
            pltpu.VMEM((BPW * 256,), jnp.float32),
            pltpu.VMEM((BPW * 128,), jnp.float32),
            pltpu.SemaphoreType.DMA,
        ],
    )(_attn_body)
    return kern(img, qh1, kp1, g1)


def kernel(sampled_points, sampled_x, edge_index_filtered, Wc, bc, Wo, bo,
           Wq, bq, Wk, bk):
    src = edge_index_filtered[0]
    dst = edge_index_filtered[1]
    qt, kall, g4, ht4 = _projections(sampled_points, sampled_x, Wq, bq,
                                     Wk, bk, Wc, bc, Wo, bo)
    qb = qt.reshape(F, NBLK, 16).transpose(1, 0, 2).reshape(NBLK, F * 16)
    hb = ht4.reshape(4, NBLK, 16).transpose(1, 0, 2).reshape(NBLK, 64)
    qh = jnp.concatenate(
        [qb, hb, jnp.zeros((NBLK, 64), jnp.float32)], axis=1)
    qh1 = jnp.concatenate(
        [qh, jnp.zeros((NBLKP - NBLK, QROW), jnp.float32)], axis=0
    ).reshape(-1)
    kp1 = kall.reshape(N, NSL, FSL).transpose(1, 0, 2).reshape(-1)
    cnt1 = _counts(src, jnp.zeros((NP,), jnp.int32))
    base_all, totals = _prefix(cnt1)
    img = _scatter(src, dst, base_all, totals,
                   jnp.zeros((NIMG,), jnp.int32))
    out1 = _attention(img, qh1, kp1, g4.reshape(-1))
    o = out1[:NBLK * 128].reshape(NBLK, 8, 16)[:, :3, :]
    return o.transpose(0, 2, 1).reshape(N, 3)
